# Initial kernel scaffold; baseline (speedup 1.0000x reference)
import functools

import jax
import jax.numpy as jnp
from jax import lax
from jax.experimental import pallas as pl
from jax.experimental.pallas import tpu as pltpu
from jax.experimental.pallas import tpu_sc as plsc

N_USERS = 25000
N_ITEMS = 25000
N = N_USERS + N_ITEMS
D = 64
H = 32
N_LAYERS = 3
E = 800000

CHUNK = 128
TILES = 16
CORES = 2
G = -(-E // (CHUNK * TILES))
E_PAD = G * CHUNK * TILES
R = E_PAD // CHUNK

NPT = N // TILES
ZROWS = 125
ZITER = NPT // ZROWS


def _body(tab0, src2d, dst2d, val1d, out_final, out_layers,
          acc, srcv, dstv, valv, rows, zbuf, m0, m1, m2, m3):
    c = lax.axis_index("c")
    s = lax.axis_index("s")

    def zinit(i, _):
        zbuf[i, pl.ds(0, 16)] = jnp.zeros((16,), jnp.float32)
        zbuf[i, pl.ds(16, 16)] = jnp.zeros((16,), jnp.float32)
        return 0
    lax.fori_loop(0, ZROWS, zinit, 0)

    for layer in range(N_LAYERS):
        tab = tab0 if layer == 0 else out_layers.at[layer - 1]

        for z in range(ZITER):
            pltpu.sync_copy(zbuf, acc.at[pl.ds(s * NPT + z * ZROWS, ZROWS)])
        plsc.subcore_barrier()

        def edge_step(g, _):
            row = s * G + g
            pltpu.sync_copy(src2d.at[pl.ds(c * R + row, 1)], srcv)
            pltpu.sync_copy(dst2d.at[pl.ds(row, 1)], dstv)
            pltpu.sync_copy(val1d.at[pl.ds(row * CHUNK, CHUNK)], valv)
            pltpu.sync_copy(tab.at[srcv.at[0]], rows)
            def scale(e, _):
                v = plsc.load_gather(valv, [jnp.full((16,), e, jnp.int32)])
                rows[e, pl.ds(0, 16)] = rows[e, pl.ds(0, 16)] * v
                rows[e, pl.ds(16, 16)] = rows[e, pl.ds(16, 16)] * v
                return 0
            lax.fori_loop(0, CHUNK, scale, 0)
            pltpu.sync_copy(rows, acc.at[dstv.at[0]], add=True)
            return 0
        lax.fori_loop(0, G, edge_step, 0)
        plsc.subcore_barrier()

        pltpu.sync_copy(acc.at[pl.ds(s * NPT, NPT)],
                        out_layers.at[layer].at[pl.ds(c * N + s * NPT, NPT)])
        plsc.subcore_barrier()

    for z in range(ZITER):
        base = c * N + s * NPT + z * ZROWS
        pltpu.sync_copy(tab0.at[pl.ds(base, ZROWS)], m0)
        pltpu.sync_copy(out_layers.at[0].at[pl.ds(base, ZROWS)], m1)
        pltpu.sync_copy(out_layers.at[1].at[pl.ds(base, ZROWS)], m2)
        pltpu.sync_copy(out_layers.at[2].at[pl.ds(base, ZROWS)], m3)

        def mean_row(i, _):
            for lo in (0, 16):
                m = (m0[i, pl.ds(lo, 16)] + m1[i, pl.ds(lo, 16)]
                     + m2[i, pl.ds(lo, 16)] + m3[i, pl.ds(lo, 16)])
                m0[i, pl.ds(lo, 16)] = m * 0.25
            return 0
        lax.fori_loop(0, ZROWS, mean_row, 0)
        pltpu.sync_copy(m0, out_final.at[pl.ds(base, ZROWS)])


@jax.jit
def _run(tab0, src2d, dst2d, val1d):
    mesh = plsc.VectorSubcoreMesh(core_axis_name="c", subcore_axis_name="s",
                                  num_cores=CORES, num_subcores=TILES)
    f = pl.kernel(
        _body,
        out_type=(
            jax.ShapeDtypeStruct((CORES * N, H), jnp.float32),
            jax.ShapeDtypeStruct((N_LAYERS, CORES * N, H), jnp.float32),
        ),
        mesh=mesh,
        scratch_types=[
            pltpu.VMEM_SHARED((N, H), jnp.float32),
            pltpu.VMEM((1, CHUNK), jnp.int32),
            pltpu.VMEM((1, CHUNK), jnp.int32),
            pltpu.VMEM((CHUNK,), jnp.float32),
            pltpu.VMEM((CHUNK, H), jnp.float32),
            pltpu.VMEM((ZROWS, H), jnp.float32),
            pltpu.VMEM((ZROWS, H), jnp.float32),
            pltpu.VMEM((ZROWS, H), jnp.float32),
            pltpu.VMEM((ZROWS, H), jnp.float32),
            pltpu.VMEM((ZROWS, H), jnp.float32),
        ],
        name="lightgcn_sc",
    )
    return f(tab0, src2d, dst2d, val1d)


def kernel(user_emb, item_emb, adj_indices, adj_values):
    emb0 = jnp.concatenate([user_emb, item_emb], axis=0)
    tab0 = jnp.concatenate([emb0[:, :H], emb0[:, H:]], axis=0)

    src = adj_indices[0].astype(jnp.int32)
    dst = adj_indices[1].astype(jnp.int32)
    val = adj_values.astype(jnp.float32)
    pad = E_PAD - E
    src = jnp.concatenate([src, jnp.zeros((pad,), jnp.int32)])
    dst = jnp.concatenate([dst, jnp.zeros((pad,), jnp.int32)])
    val = jnp.concatenate([val, jnp.zeros((pad,), jnp.float32)])

    src2d = jnp.concatenate([src, src + N]).reshape(2 * R, CHUNK)
    dst2d = dst.reshape(R, CHUNK)

    final, _ = _run(tab0, src2d, dst2d, val)
    full = jnp.concatenate([final[:N], final[N:]], axis=1)
    return (full[:N_USERS], full[N_USERS:])


# speedup vs baseline: 5.5604x; 5.5604x over previous
import jax
import jax.numpy as jnp
from jax import lax
from jax.experimental import pallas as pl
from jax.experimental.pallas import tpu as pltpu
from jax.experimental.pallas import tpu_sc as plsc

N_USERS = 25000
N_ITEMS = 25000
N = N_USERS + N_ITEMS
H = 32
N_LAYERS = 3
E = 800000

CHUNK = 128
GRP = 8
TILES = 16
CORES = 2
GG = -(-E // (CHUNK * GRP * TILES))
G = GG * GRP
E_PAD = G * CHUNK * TILES
R = E_PAD // CHUNK

NPT = 3128
N_PAD = NPT * TILES
ZROWS = 136
ZITER = NPT // ZROWS


def _body(tab0, src2d, dst2d, val1d, out_final, out_layers,
          acc, srcv, dstv, valv, rows, zbuf, m0, m1, m2, m3):
    c = lax.axis_index("c")
    s = lax.axis_index("s")

    def zinit(i, _):
        zbuf[i, pl.ds(0, 16)] = jnp.zeros((16,), jnp.float32)
        zbuf[i, pl.ds(16, 16)] = jnp.zeros((16,), jnp.float32)
        return 0
    lax.fori_loop(0, ZROWS, zinit, 0)

    for layer in range(N_LAYERS):
        tab = tab0 if layer == 0 else out_layers.at[layer - 1]

        for z in range(ZITER):
            pltpu.sync_copy(zbuf, acc.at[pl.ds(s * NPT + z * ZROWS, ZROWS)])
        plsc.subcore_barrier()

        def edge_group(g, _):
            row = s * G + g * GRP
            pltpu.sync_copy(src2d.at[pl.ds(c * R + row, GRP)], srcv)
            pltpu.sync_copy(dst2d.at[pl.ds(row, GRP)], dstv)
            pltpu.sync_copy(val1d.at[pl.ds(row * CHUNK, GRP * CHUNK)], valv)
            for j in range(GRP):
                pltpu.sync_copy(tab.at[srcv.at[j]], rows)

                def scale(q, _):
                    vv = valv[pl.ds((j * 8 + q) * 16, 16)]
                    for e16 in range(16):
                        v = lax.gather(
                            vv, jnp.full((16, 1), e16, jnp.int32),
                            lax.GatherDimensionNumbers(
                                offset_dims=(), collapsed_slice_dims=(0,),
                                start_index_map=(0,)),
                            (1,),
                            mode=lax.GatherScatterMode.PROMISE_IN_BOUNDS)
                        e = q * 16 + e16
                        rows[e, pl.ds(0, 16)] = rows[e, pl.ds(0, 16)] * v
                        rows[e, pl.ds(16, 16)] = rows[e, pl.ds(16, 16)] * v
                    return 0
                lax.fori_loop(0, CHUNK // 16, scale, 0)
                pltpu.sync_copy(rows, acc.at[dstv.at[j]], add=True)
            return 0
        lax.fori_loop(0, GG, edge_group, 0)
        plsc.subcore_barrier()

        pltpu.sync_copy(acc.at[pl.ds(s * NPT, NPT)],
                        out_layers.at[layer].at[pl.ds(c * N_PAD + s * NPT, NPT)])
        plsc.subcore_barrier()

    for z in range(ZITER):
        base = c * N_PAD + s * NPT + z * ZROWS
        pltpu.sync_copy(tab0.at[pl.ds(base, ZROWS)], m0)
        pltpu.sync_copy(out_layers.at[0].at[pl.ds(base, ZROWS)], m1)
        pltpu.sync_copy(out_layers.at[1].at[pl.ds(base, ZROWS)], m2)
        pltpu.sync_copy(out_layers.at[2].at[pl.ds(base, ZROWS)], m3)

        def mean_row(i, _):
            for lo in (0, 16):
                m = (m0[i, pl.ds(lo, 16)] + m1[i, pl.ds(lo, 16)]
                     + m2[i, pl.ds(lo, 16)] + m3[i, pl.ds(lo, 16)])
                m0[i, pl.ds(lo, 16)] = m * 0.25
            return 0
        lax.fori_loop(0, ZROWS, mean_row, 0)
        pltpu.sync_copy(m0, out_final.at[pl.ds(base, ZROWS)])


@jax.jit
def _run(tab0, src2d, dst2d, val1d):
    mesh = plsc.VectorSubcoreMesh(core_axis_name="c", subcore_axis_name="s",
                                  num_cores=CORES, num_subcores=TILES)
    f = pl.kernel(
        _body,
        out_type=(
            jax.ShapeDtypeStruct((CORES * N_PAD, H), jnp.float32),
            jax.ShapeDtypeStruct((N_LAYERS, CORES * N_PAD, H), jnp.float32),
        ),
        mesh=mesh,
        scratch_types=[
            pltpu.VMEM_SHARED((N_PAD, H), jnp.float32),
            pltpu.VMEM((GRP, CHUNK), jnp.int32),
            pltpu.VMEM((GRP, CHUNK), jnp.int32),
            pltpu.VMEM((GRP * CHUNK,), jnp.float32),
            pltpu.VMEM((CHUNK, H), jnp.float32),
            pltpu.VMEM((ZROWS, H), jnp.float32),
            pltpu.VMEM((ZROWS, H), jnp.float32),
            pltpu.VMEM((ZROWS, H), jnp.float32),
            pltpu.VMEM((ZROWS, H), jnp.float32),
            pltpu.VMEM((ZROWS, H), jnp.float32),
        ],
        compiler_params=pltpu.CompilerParams(use_tc_tiling_on_sc=False),
        name="lightgcn_sc",
    )
    return f(tab0, src2d, dst2d, val1d)


def kernel(user_emb, item_emb, adj_indices, adj_values):
    emb0 = jnp.concatenate([user_emb, item_emb], axis=0)
    npad = N_PAD - N
    zrows = jnp.zeros((npad, H), jnp.float32)
    tab0 = jnp.concatenate([emb0[:, :H], zrows, emb0[:, H:], zrows], axis=0)

    src = adj_indices[0].astype(jnp.int32)
    dst = adj_indices[1].astype(jnp.int32)
    val = adj_values.astype(jnp.float32)
    pad = E_PAD - E
    src = jnp.concatenate([src, jnp.zeros((pad,), jnp.int32)])
    dst = jnp.concatenate([dst, jnp.zeros((pad,), jnp.int32)])
    val = jnp.concatenate([val, jnp.zeros((pad,), jnp.float32)])

    src2d = jnp.concatenate([src, src + N_PAD]).reshape(2 * R, CHUNK)
    dst2d = dst.reshape(R, CHUNK)

    final, _ = _run(tab0, src2d, dst2d, val)
    full = jnp.concatenate([final[:N], final[N_PAD:N_PAD + N]], axis=1)
    return (full[:N_USERS], full[N_USERS:])


# speedup vs baseline: 9.8372x; 1.7691x over previous
import jax
import jax.numpy as jnp
from jax import lax
from jax.experimental import pallas as pl
from jax.experimental.pallas import tpu as pltpu
from jax.experimental.pallas import tpu_sc as plsc

N_USERS = 25000
N_ITEMS = 25000
N = N_USERS + N_ITEMS
H = 32
N_LAYERS = 3
E = 800000

CHUNK = 128
BROWS = 8
EPB = BROWS * CHUNK
TILES = 16
CORES = 2
NBLK = 49
G = NBLK * BROWS
E_PAD = G * CHUNK * TILES
R = E_PAD // CHUNK

NPT = 3200
N_PAD = NPT * TILES
ZROWS = 64
ZITER = NPT // ZROWS


def _body(tab0, src2d, dst2d, val1d, out_final, out_layers,
          acc, srcv, dstv, valv, rows2, ma, mb, mc, md,
          gsem, ssem, isem, msem):
    c = lax.axis_index("c")
    s = lax.axis_index("s")

    def zinit(i, _):
        ma[i, pl.ds(0, 16)] = jnp.zeros((16,), jnp.float32)
        ma[i, pl.ds(16, 16)] = jnp.zeros((16,), jnp.float32)
        return 0
    lax.fori_loop(0, ZROWS, zinit, 0)

    def idx_block_copies(bl, slot, copy_fn):
        row = s * G + bl * BROWS
        out = []
        out.append(copy_fn(src2d.at[pl.ds(c * R + row, BROWS)],
                           srcv.at[slot], isem))
        out.append(copy_fn(dst2d.at[pl.ds(row, BROWS)], dstv.at[slot], isem))
        out.append(copy_fn(val1d.at[pl.ds(row * CHUNK, EPB)],
                           valv.at[slot], isem))
        return out

    for layer in range(N_LAYERS):
        tab = tab0 if layer == 0 else out_layers.at[layer - 1]

        def fire_gather(idx_slot, j, row_slot):
            pltpu.async_copy(tab.at[srcv.at[idx_slot].at[j]],
                             rows2.at[row_slot], gsem.at[row_slot])

        def wait_gather(idx_slot, j, row_slot):
            pltpu.make_async_copy(tab.at[srcv.at[idx_slot].at[j]],
                                  rows2.at[row_slot],
                                  gsem.at[row_slot]).wait()

        def fire_scatter(idx_slot, j, row_slot):
            pltpu.async_copy(rows2.at[row_slot],
                             acc.at[dstv.at[idx_slot].at[j]],
                             ssem.at[row_slot], add=True)

        def wait_scatter(idx_slot, j, row_slot):
            pltpu.make_async_copy(rows2.at[row_slot],
                                  acc.at[dstv.at[idx_slot].at[j]],
                                  ssem.at[row_slot]).wait()

        for z in range(ZITER):
            pltpu.sync_copy(ma, acc.at[pl.ds(s * NPT + z * ZROWS, ZROWS)])
        plsc.subcore_barrier()

        def sync3(src, dst, sem):
            pltpu.sync_copy(src, dst)
        idx_block_copies(0, 0, sync3)
        fire_gather(0, 0, 0)

        def block(bl, _):
            cb = bl % 2
            nb = 1 - cb
            for j in range(BROWS):
                sj = j % 2
                pj = 1 - sj
                if j == 0:
                    @pl.when(bl >= 1)
                    def _():
                        wait_scatter(nb, BROWS - 1, 1)
                    @pl.when(bl < NBLK - 1)
                    def _():
                        idx_block_copies(bl + 1, nb, pltpu.async_copy)
                else:
                    wait_scatter(cb, j - 1, pj)
                if j < BROWS - 1:
                    fire_gather(cb, j + 1, pj)
                else:
                    @pl.when(bl < NBLK - 1)
                    def _():
                        for d in idx_block_copies(bl + 1, nb,
                                                  pltpu.make_async_copy):
                            d.wait()
                        fire_gather(nb, 0, pj)
                wait_gather(cb, j, sj)

                def scale(q, _):
                    vv = valv[cb, pl.ds(j * CHUNK + q * 16, 16)]
                    for e16 in range(16):
                        v = lax.gather(
                            vv, jnp.full((16, 1), e16, jnp.int32),
                            lax.GatherDimensionNumbers(
                                offset_dims=(), collapsed_slice_dims=(0,),
                                start_index_map=(0,)),
                            (1,),
                            mode=lax.GatherScatterMode.PROMISE_IN_BOUNDS)
                        e = q * 16 + e16
                        rows2[sj, e, pl.ds(0, 16)] = (
                            rows2[sj, e, pl.ds(0, 16)] * v)
                        rows2[sj, e, pl.ds(16, 16)] = (
                            rows2[sj, e, pl.ds(16, 16)] * v)
                    return 0
                lax.fori_loop(0, CHUNK // 16, scale, 0)

                fire_scatter(cb, j, sj)
            return 0
        lax.fori_loop(0, NBLK, block, 0)
        wait_scatter(0, BROWS - 1, 1)
        plsc.subcore_barrier()

        pltpu.sync_copy(acc.at[pl.ds(s * NPT, NPT)],
                        out_layers.at[layer].at[pl.ds(c * N_PAD + s * NPT, NPT)])
        plsc.subcore_barrier()

    for z in range(ZITER):
        base = c * N_PAD + s * NPT + z * ZROWS
        d0 = pltpu.async_copy(tab0.at[pl.ds(base, ZROWS)], ma, msem)
        d1 = pltpu.async_copy(out_layers.at[0].at[pl.ds(base, ZROWS)], mb, msem)
        d2 = pltpu.async_copy(out_layers.at[1].at[pl.ds(base, ZROWS)], mc, msem)
        d3 = pltpu.async_copy(out_layers.at[2].at[pl.ds(base, ZROWS)], md, msem)
        d0.wait(); d1.wait(); d2.wait(); d3.wait()

        def mean_row(i, _):
            for lo in (0, 16):
                m = (ma[i, pl.ds(lo, 16)] + mb[i, pl.ds(lo, 16)]
                     + mc[i, pl.ds(lo, 16)] + md[i, pl.ds(lo, 16)])
                ma[i, pl.ds(lo, 16)] = m * 0.25
            return 0
        lax.fori_loop(0, ZROWS, mean_row, 0)
        pltpu.sync_copy(ma, out_final.at[pl.ds(base, ZROWS)])


@jax.jit
def _run(tab0, src2d, dst2d, val1d):
    mesh = plsc.VectorSubcoreMesh(core_axis_name="c", subcore_axis_name="s",
                                  num_cores=CORES, num_subcores=TILES)
    f = pl.kernel(
        _body,
        out_type=(
            jax.ShapeDtypeStruct((CORES * N_PAD, H), jnp.float32),
            jax.ShapeDtypeStruct((N_LAYERS, CORES * N_PAD, H), jnp.float32),
        ),
        mesh=mesh,
        scratch_types=[
            pltpu.VMEM_SHARED((N_PAD, H), jnp.float32),
            pltpu.VMEM((2, BROWS, CHUNK), jnp.int32),
            pltpu.VMEM((2, BROWS, CHUNK), jnp.int32),
            pltpu.VMEM((2, EPB), jnp.float32),
            pltpu.VMEM((2, CHUNK, H), jnp.float32),
            pltpu.VMEM((ZROWS, H), jnp.float32),
            pltpu.VMEM((ZROWS, H), jnp.float32),
            pltpu.VMEM((ZROWS, H), jnp.float32),
            pltpu.VMEM((ZROWS, H), jnp.float32),
            pltpu.SemaphoreType.DMA((2,)),
            pltpu.SemaphoreType.DMA((2,)),
            pltpu.SemaphoreType.DMA,
            pltpu.SemaphoreType.DMA,
        ],
        compiler_params=pltpu.CompilerParams(use_tc_tiling_on_sc=False),
        name="lightgcn_sc",
    )
    return f(tab0, src2d, dst2d, val1d)


def kernel(user_emb, item_emb, adj_indices, adj_values):
    emb0 = jnp.concatenate([user_emb, item_emb], axis=0)
    npad = N_PAD - N
    zrows = jnp.zeros((npad, H), jnp.float32)
    tab0 = jnp.concatenate([emb0[:, :H], zrows, emb0[:, H:], zrows], axis=0)

    src = adj_indices[0].astype(jnp.int32)
    dst = adj_indices[1].astype(jnp.int32)
    val = adj_values.astype(jnp.float32)
    pad = E_PAD - E
    src = jnp.concatenate([src, jnp.zeros((pad,), jnp.int32)])
    dst = jnp.concatenate([dst, jnp.zeros((pad,), jnp.int32)])
    val = jnp.concatenate([val, jnp.zeros((pad,), jnp.float32)])

    src2d = jnp.concatenate([src, src + N_PAD]).reshape(2 * R, CHUNK)
    dst2d = dst.reshape(R, CHUNK)

    final, _ = _run(tab0, src2d, dst2d, val)
    full = jnp.concatenate([final[:N], final[N_PAD:N_PAD + N]], axis=1)
    return (full[:N_USERS], full[N_USERS:])


# speedup vs baseline: 12.2074x; 1.2409x over previous
import jax
import jax.numpy as jnp
from jax import lax
from jax.experimental import pallas as pl
from jax.experimental.pallas import tpu as pltpu
from jax.experimental.pallas import tpu_sc as plsc

N_USERS = 25000
N_ITEMS = 25000
N = N_USERS + N_ITEMS
H = 32
N_LAYERS = 3
E = 800000

CHUNK = 128
BROWS = 8
EPB = BROWS * CHUNK
NSLOT = 4
DEPTH = 2
TILES = 16
CORES = 2
NBLK = 49
G = NBLK * BROWS
E_PAD = G * CHUNK * TILES
R = E_PAD // CHUNK

NPT = 3200
N_PAD = NPT * TILES
ZROWS = 32
ZITER = NPT // ZROWS
ZWAVE = 5


def _body(tab0, src2d, dst2d, val1d, out_final, out_layers,
          acc, srcv, dstv, valv, rows4, ma, mb, mc, md,
          gsem, ssem, isem, msem):
    c = lax.axis_index("c")
    s = lax.axis_index("s")

    def zinit(i, _):
        ma[i, pl.ds(0, 16)] = jnp.zeros((16,), jnp.float32)
        ma[i, pl.ds(16, 16)] = jnp.zeros((16,), jnp.float32)
        return 0
    lax.fori_loop(0, ZROWS, zinit, 0)

    def idx_block_copies(bl, slot, copy_fn):
        row = s * G + bl * BROWS
        out = []
        out.append(copy_fn(src2d.at[pl.ds(c * R + row, BROWS)],
                           srcv.at[slot], isem))
        out.append(copy_fn(dst2d.at[pl.ds(row, BROWS)], dstv.at[slot], isem))
        out.append(copy_fn(val1d.at[pl.ds(row * CHUNK, EPB)],
                           valv.at[slot], isem))
        return out

    def zero_stripe():
        def wave(w, _):
            base = s * NPT + w * ZWAVE * ZROWS
            for i in range(ZWAVE):
                pltpu.async_copy(ma, acc.at[pl.ds(base + i * ZROWS, ZROWS)],
                                 msem)
            for i in range(ZWAVE):
                pltpu.make_async_copy(
                    ma, acc.at[pl.ds(base + i * ZROWS, ZROWS)], msem).wait()
            return 0
        lax.fori_loop(0, ZITER // ZWAVE, wave, 0)

    for layer in range(N_LAYERS):
        tab = tab0 if layer == 0 else out_layers.at[layer - 1]

        def fire_gather(idx_slot, j, row_slot):
            pltpu.async_copy(tab.at[srcv.at[idx_slot].at[j]],
                             rows4.at[row_slot], gsem.at[row_slot])

        def wait_gather(idx_slot, j, row_slot):
            pltpu.make_async_copy(tab.at[srcv.at[idx_slot].at[j]],
                                  rows4.at[row_slot],
                                  gsem.at[row_slot]).wait()

        def fire_scatter(idx_slot, j, row_slot):
            pltpu.async_copy(rows4.at[row_slot],
                             acc.at[dstv.at[idx_slot].at[j]],
                             ssem.at[row_slot], add=True)

        def wait_scatter(idx_slot, j, row_slot):
            pltpu.make_async_copy(rows4.at[row_slot],
                                  acc.at[dstv.at[idx_slot].at[j]],
                                  ssem.at[row_slot]).wait()

        zero_stripe()
        plsc.subcore_barrier()

        def sync3(src, dst, sem):
            pltpu.sync_copy(src, dst)
        idx_block_copies(0, 0, sync3)
        for j0 in range(DEPTH):
            fire_gather(0, j0, j0)

        def block(bl, _):
            cb = bl % 2
            nb = 1 - cb
            for j in range(BROWS):
                sj = j % NSLOT
                fs = (j + DEPTH) % NSLOT

                if j == 0:
                    @pl.when(bl >= 1)
                    def _():
                        wait_scatter(nb, BROWS - 2, (BROWS - 2) % NSLOT)
                        wait_scatter(nb, BROWS - 1, (BROWS - 1) % NSLOT)
                    @pl.when(bl < NBLK - 1)
                    def _():
                        idx_block_copies(bl + 1, nb, pltpu.async_copy)
                elif j >= DEPTH:
                    wait_scatter(cb, j - DEPTH, (j - DEPTH) % NSLOT)

                if j + DEPTH < BROWS:
                    fire_gather(cb, j + DEPTH, fs)
                else:
                    @pl.when(bl < NBLK - 1)
                    def _():
                        if j == BROWS - DEPTH:
                            for d in idx_block_copies(bl + 1, nb,
                                                      pltpu.make_async_copy):
                                d.wait()
                        fire_gather(nb, j + DEPTH - BROWS, fs)

                wait_gather(cb, j, sj)

                def scale(q, _):
                    vv = valv[cb, pl.ds(j * CHUNK + q * 16, 16)]
                    for e16 in range(16):
                        v = lax.gather(
                            vv, jnp.full((16, 1), e16, jnp.int32),
                            lax.GatherDimensionNumbers(
                                offset_dims=(), collapsed_slice_dims=(0,),
                                start_index_map=(0,)),
                            (1,),
                            mode=lax.GatherScatterMode.PROMISE_IN_BOUNDS)
                        e = q * 16 + e16
                        rows4[sj, e, pl.ds(0, 16)] = (
                            rows4[sj, e, pl.ds(0, 16)] * v)
                        rows4[sj, e, pl.ds(16, 16)] = (
                            rows4[sj, e, pl.ds(16, 16)] * v)
                    return 0
                lax.fori_loop(0, CHUNK // 16, scale, 0)

                fire_scatter(cb, j, sj)
            return 0
        lax.fori_loop(0, NBLK, block, 0)
        wait_scatter(0, BROWS - 2, (BROWS - 2) % NSLOT)
        wait_scatter(0, BROWS - 1, (BROWS - 1) % NSLOT)
        plsc.subcore_barrier()

        pltpu.sync_copy(acc.at[pl.ds(s * NPT, NPT)],
                        out_layers.at[layer].at[pl.ds(c * N_PAD + s * NPT, NPT)])
        plsc.subcore_barrier()

    def mean_chunk(z, _):
        base = c * N_PAD + s * NPT + z * ZROWS
        d0 = pltpu.async_copy(tab0.at[pl.ds(base, ZROWS)], ma, msem)
        d1 = pltpu.async_copy(out_layers.at[0].at[pl.ds(base, ZROWS)], mb, msem)
        d2 = pltpu.async_copy(out_layers.at[1].at[pl.ds(base, ZROWS)], mc, msem)
        d3 = pltpu.async_copy(out_layers.at[2].at[pl.ds(base, ZROWS)], md, msem)
        d0.wait(); d1.wait(); d2.wait(); d3.wait()

        def mean_row(i, _):
            for lo in (0, 16):
                m = (ma[i, pl.ds(lo, 16)] + mb[i, pl.ds(lo, 16)]
                     + mc[i, pl.ds(lo, 16)] + md[i, pl.ds(lo, 16)])
                ma[i, pl.ds(lo, 16)] = m * 0.25
            return 0
        lax.fori_loop(0, ZROWS, mean_row, 0)
        pltpu.sync_copy(ma, out_final.at[pl.ds(base, ZROWS)])
        return 0
    lax.fori_loop(0, ZITER, mean_chunk, 0)


@jax.jit
def _run(tab0, src2d, dst2d, val1d):
    mesh = plsc.VectorSubcoreMesh(core_axis_name="c", subcore_axis_name="s",
                                  num_cores=CORES, num_subcores=TILES)
    f = pl.kernel(
        _body,
        out_type=(
            jax.ShapeDtypeStruct((CORES * N_PAD, H), jnp.float32),
            jax.ShapeDtypeStruct((N_LAYERS, CORES * N_PAD, H), jnp.float32),
        ),
        mesh=mesh,
        scratch_types=[
            pltpu.VMEM_SHARED((N_PAD, H), jnp.float32),
            pltpu.VMEM((2, BROWS, CHUNK), jnp.int32),
            pltpu.VMEM((2, BROWS, CHUNK), jnp.int32),
            pltpu.VMEM((2, EPB), jnp.float32),
            pltpu.VMEM((NSLOT, CHUNK, H), jnp.float32),
            pltpu.VMEM((ZROWS, H), jnp.float32),
            pltpu.VMEM((ZROWS, H), jnp.float32),
            pltpu.VMEM((ZROWS, H), jnp.float32),
            pltpu.VMEM((ZROWS, H), jnp.float32),
            pltpu.SemaphoreType.DMA((NSLOT,)),
            pltpu.SemaphoreType.DMA((NSLOT,)),
            pltpu.SemaphoreType.DMA,
            pltpu.SemaphoreType.DMA,
        ],
        compiler_params=pltpu.CompilerParams(use_tc_tiling_on_sc=False),
        name="lightgcn_sc",
    )
    return f(tab0, src2d, dst2d, val1d)


def kernel(user_emb, item_emb, adj_indices, adj_values):
    emb0 = jnp.concatenate([user_emb, item_emb], axis=0)
    npad = N_PAD - N
    zrows = jnp.zeros((npad, H), jnp.float32)
    tab0 = jnp.concatenate([emb0[:, :H], zrows, emb0[:, H:], zrows], axis=0)

    src = adj_indices[0].astype(jnp.int32)
    dst = adj_indices[1].astype(jnp.int32)
    val = adj_values.astype(jnp.float32)
    pad = E_PAD - E
    src = jnp.concatenate([src, jnp.zeros((pad,), jnp.int32)])
    dst = jnp.concatenate([dst, jnp.zeros((pad,), jnp.int32)])
    val = jnp.concatenate([val, jnp.zeros((pad,), jnp.float32)])

    src2d = jnp.concatenate([src, src + N_PAD]).reshape(2 * R, CHUNK)
    dst2d = dst.reshape(R, CHUNK)

    final, _ = _run(tab0, src2d, dst2d, val)
    full = jnp.concatenate([final[:N], final[N_PAD:N_PAD + N]], axis=1)
    return (full[:N_USERS], full[N_USERS:])


# speedup vs baseline: 13.6890x; 1.1214x over previous
import jax
import jax.numpy as jnp
from jax import lax
from jax.experimental import pallas as pl
from jax.experimental.pallas import tpu as pltpu
from jax.experimental.pallas import tpu_sc as plsc

N_USERS = 25000
N_ITEMS = 25000
N = N_USERS + N_ITEMS
H = 32
N_LAYERS = 3
E = 800000

CHUNK = 128
BROWS = 8
EPB = BROWS * CHUNK
NSLOT = 4
DEPTH = 2
TILES = 16
CORES = 2
NBLK = 49
G = NBLK * BROWS
E_PAD = G * CHUNK * TILES
R = E_PAD // CHUNK

NPT = 3200
N_PAD = NPT * TILES
ZROWS = 40
ZITER = NPT // ZROWS
ZWAVE = 5
MCHUNKS = N_USERS // ZROWS


def _body(tab0, src2d, dst2d, val2d, out_users, out_items, out_layers,
          acc, srcv, dstv, valv, rows4, ma, mb, mc, md,
          gsem, ssem, isem, msem):
    c = lax.axis_index("c")
    s = lax.axis_index("s")

    def zinit(i, _):
        ma[i, pl.ds(0, 16)] = jnp.zeros((16,), jnp.float32)
        ma[i, pl.ds(16, 16)] = jnp.zeros((16,), jnp.float32)
        return 0
    lax.fori_loop(0, ZROWS, zinit, 0)

    def idx_block_copies(bl, slot, copy_fn):
        row = s * G + bl * BROWS
        out = []
        out.append(copy_fn(src2d.at[pl.ds(row, BROWS)], srcv.at[slot], isem))
        out.append(copy_fn(dst2d.at[pl.ds(row, BROWS)], dstv.at[slot], isem))
        out.append(copy_fn(val2d.at[pl.ds(row, BROWS)], valv.at[slot], isem))
        return out

    def offset_src(slot):
        off = jnp.full((16,), c * N_PAD, jnp.int32)
        for r in range(BROWS):
            def add16(k, _):
                srcv[slot, r, pl.ds(k * 16, 16)] = (
                    srcv[slot, r, pl.ds(k * 16, 16)] + off)
                return 0
            lax.fori_loop(0, CHUNK // 16, add16, 0)

    def zero_stripe():
        def wave(w, _):
            base = s * NPT + w * ZWAVE * ZROWS
            for i in range(ZWAVE):
                pltpu.async_copy(ma, acc.at[pl.ds(base + i * ZROWS, ZROWS)],
                                 msem)
            for i in range(ZWAVE):
                pltpu.make_async_copy(
                    ma, acc.at[pl.ds(base + i * ZROWS, ZROWS)], msem).wait()
            return 0
        lax.fori_loop(0, ZITER // ZWAVE, wave, 0)

    for layer in range(N_LAYERS):
        tab = tab0 if layer == 0 else out_layers.at[layer - 1]

        def fire_gather(idx_slot, j, row_slot):
            pltpu.async_copy(tab.at[srcv.at[idx_slot].at[j]],
                             rows4.at[row_slot], gsem.at[row_slot])

        def wait_gather(idx_slot, j, row_slot):
            pltpu.make_async_copy(tab.at[srcv.at[idx_slot].at[j]],
                                  rows4.at[row_slot],
                                  gsem.at[row_slot]).wait()

        def fire_scatter(idx_slot, j, row_slot):
            pltpu.async_copy(rows4.at[row_slot],
                             acc.at[dstv.at[idx_slot].at[j]],
                             ssem.at[row_slot], add=True)

        def wait_scatter(idx_slot, j, row_slot):
            pltpu.make_async_copy(rows4.at[row_slot],
                                  acc.at[dstv.at[idx_slot].at[j]],
                                  ssem.at[row_slot]).wait()

        zero_stripe()
        plsc.subcore_barrier()

        def sync3(src, dst, sem):
            pltpu.sync_copy(src, dst)
        idx_block_copies(0, 0, sync3)
        offset_src(0)
        for j0 in range(DEPTH):
            fire_gather(0, j0, j0)

        def block(bl, _):
            cb = bl % 2
            nb = 1 - cb
            for j in range(BROWS):
                sj = j % NSLOT
                fs = (j + DEPTH) % NSLOT

                if j == 0:
                    @pl.when(bl >= 1)
                    def _():
                        wait_scatter(nb, BROWS - 2, (BROWS - 2) % NSLOT)
                        wait_scatter(nb, BROWS - 1, (BROWS - 1) % NSLOT)
                    @pl.when(bl < NBLK - 1)
                    def _():
                        idx_block_copies(bl + 1, nb, pltpu.async_copy)
                elif j >= DEPTH:
                    wait_scatter(cb, j - DEPTH, (j - DEPTH) % NSLOT)

                if j + DEPTH < BROWS:
                    fire_gather(cb, j + DEPTH, fs)
                else:
                    @pl.when(bl < NBLK - 1)
                    def _():
                        if j == BROWS - DEPTH:
                            for d in idx_block_copies(bl + 1, nb,
                                                      pltpu.make_async_copy):
                                d.wait()
                            offset_src(nb)
                        fire_gather(nb, j + DEPTH - BROWS, fs)

                wait_gather(cb, j, sj)

                def scale(q, _):
                    vv = valv[cb, j, pl.ds(q * 16, 16)]
                    for e16 in range(16):
                        v = lax.gather(
                            vv, jnp.full((16, 1), e16, jnp.int32),
                            lax.GatherDimensionNumbers(
                                offset_dims=(), collapsed_slice_dims=(0,),
                                start_index_map=(0,)),
                            (1,),
                            mode=lax.GatherScatterMode.PROMISE_IN_BOUNDS)
                        e = q * 16 + e16
                        rows4[sj, e, pl.ds(0, 16)] = (
                            rows4[sj, e, pl.ds(0, 16)] * v)
                        rows4[sj, e, pl.ds(16, 16)] = (
                            rows4[sj, e, pl.ds(16, 16)] * v)
                    return 0
                lax.fori_loop(0, CHUNK // 16, scale, 0)

                fire_scatter(cb, j, sj)
            return 0
        lax.fori_loop(0, NBLK, block, 0)
        wait_scatter(0, BROWS - 2, (BROWS - 2) % NSLOT)
        wait_scatter(0, BROWS - 1, (BROWS - 1) % NSLOT)
        plsc.subcore_barrier()

        pltpu.sync_copy(acc.at[pl.ds(s * NPT, NPT)],
                        out_layers.at[layer].at[pl.ds(c * N_PAD + s * NPT, NPT)])
        plsc.subcore_barrier()

    def mean_table(node_off, out_ref):
        nk = jnp.where(s == 0, (MCHUNKS + TILES - 1) // TILES,
                       MCHUNKS // TILES)

        def mean_chunk(k, _):
            q = s + k * TILES
            base = c * N_PAD + node_off + q * ZROWS
            d0 = pltpu.async_copy(tab0.at[pl.ds(base, ZROWS)], ma, msem)
            d1 = pltpu.async_copy(out_layers.at[0].at[pl.ds(base, ZROWS)],
                                  mb, msem)
            d2 = pltpu.async_copy(out_layers.at[1].at[pl.ds(base, ZROWS)],
                                  mc, msem)
            d3 = pltpu.async_copy(out_layers.at[2].at[pl.ds(base, ZROWS)],
                                  md, msem)
            d0.wait(); d1.wait(); d2.wait(); d3.wait()

            def mean_row(i, _):
                for lo in (0, 16):
                    m = (ma[i, pl.ds(lo, 16)] + mb[i, pl.ds(lo, 16)]
                         + mc[i, pl.ds(lo, 16)] + md[i, pl.ds(lo, 16)])
                    ma[i, pl.ds(lo, 16)] = m * 0.25
                return 0
            lax.fori_loop(0, ZROWS, mean_row, 0)
            pltpu.sync_copy(ma, out_ref.at[pl.ds(q * ZROWS, ZROWS),
                                           pl.ds(c * H, H)])
            return 0
        lax.fori_loop(0, nk, mean_chunk, 0)

    mean_table(0, out_users)
    mean_table(N_USERS, out_items)


@jax.jit
def _run(tab0, src2d, dst2d, val2d):
    mesh = plsc.VectorSubcoreMesh(core_axis_name="c", subcore_axis_name="s",
                                  num_cores=CORES, num_subcores=TILES)
    f = pl.kernel(
        _body,
        out_type=(
            jax.ShapeDtypeStruct((N_USERS, 2 * H), jnp.float32),
            jax.ShapeDtypeStruct((N_ITEMS, 2 * H), jnp.float32),
            jax.ShapeDtypeStruct((N_LAYERS, CORES * N_PAD, H), jnp.float32),
        ),
        mesh=mesh,
        scratch_types=[
            pltpu.VMEM_SHARED((N_PAD, H), jnp.float32),
            pltpu.VMEM((2, BROWS, CHUNK), jnp.int32),
            pltpu.VMEM((2, BROWS, CHUNK), jnp.int32),
            pltpu.VMEM((2, BROWS, CHUNK), jnp.float32),
            pltpu.VMEM((NSLOT, CHUNK, H), jnp.float32),
            pltpu.VMEM((ZROWS, H), jnp.float32),
            pltpu.VMEM((ZROWS, H), jnp.float32),
            pltpu.VMEM((ZROWS, H), jnp.float32),
            pltpu.VMEM((ZROWS, H), jnp.float32),
            pltpu.SemaphoreType.DMA((NSLOT,)),
            pltpu.SemaphoreType.DMA((NSLOT,)),
            pltpu.SemaphoreType.DMA,
            pltpu.SemaphoreType.DMA,
        ],
        compiler_params=pltpu.CompilerParams(use_tc_tiling_on_sc=False),
        name="lightgcn_sc",
    )
    return f(tab0, src2d, dst2d, val2d)


def kernel(user_emb, item_emb, adj_indices, adj_values):
    emb0 = jnp.concatenate([user_emb, item_emb], axis=0)
    npad = N_PAD - N
    zrows = jnp.zeros((npad, H), jnp.float32)
    tab0 = jnp.concatenate([emb0[:, :H], zrows, emb0[:, H:], zrows], axis=0)

    pad = E_PAD - E
    src = jnp.concatenate([adj_indices[0].astype(jnp.int32),
                           jnp.zeros((pad,), jnp.int32)])
    dst = jnp.concatenate([adj_indices[1].astype(jnp.int32),
                           jnp.zeros((pad,), jnp.int32)])
    val = jnp.concatenate([adj_values.astype(jnp.float32),
                           jnp.zeros((pad,), jnp.float32)])

    users, items, _ = _run(tab0, src.reshape(R, CHUNK),
                           dst.reshape(R, CHUNK), val.reshape(R, CHUNK))
    return (users, items)
